# baseline (speedup 1.0000x reference)
import functools

import jax
import jax.numpy as jnp
from jax import lax
from jax.experimental import pallas as pl
from jax.experimental.pallas import tpu as pltpu

NUM_CLASSES = 80
MAX_PER_IMG = 100
MASK_THR = 0.5
ORI_H = 512
ORI_W = 512
IN_HW = 128
MASKS_PER_STEP = 4
NEG_INF = float("-inf")
SLAB = 64
SLAB_STARTS = (0, 16, 48, 64)


def _topk_body(scores_ref, vals_ref, labels_ref, src_ref):
    b, n = scores_ref.shape
    scores0 = scores_ref[...]
    iota = lax.broadcasted_iota(jnp.int32, (b, n), 1)
    out_iota = lax.broadcasted_iota(jnp.int32, (b, 128), 1)

    def step(k, carry):
        scores, vals_acc, idx_acc = carry
        m = jnp.max(scores, axis=1, keepdims=True)
        cand = jnp.where(scores == m, iota, jnp.int32(2**30))
        idx = jnp.min(cand, axis=1, keepdims=True)
        vals_acc = jnp.where(out_iota == k, m, vals_acc)
        idx_acc = jnp.where(out_iota == k, idx, idx_acc)
        scores = jnp.where(iota == idx, NEG_INF, scores)
        return scores, vals_acc, idx_acc

    init = (
        scores0,
        jnp.zeros((b, 128), jnp.float32),
        jnp.zeros((b, 128), jnp.int32),
    )
    _, vals_acc, idx_acc = lax.fori_loop(0, MAX_PER_IMG, step, init)

    row = lax.broadcasted_iota(jnp.int32, (b, 128), 0)
    vals_ref[...] = vals_acc
    labels_ref[...] = idx_acc % NUM_CLASSES
    src_ref[...] = idx_acc // NUM_CLASSES + MAX_PER_IMG * row


def _split3(a):
    hi = a.astype(jnp.bfloat16)
    r1 = a - hi.astype(jnp.float32)
    mid = r1.astype(jnp.bfloat16)
    lo = (r1 - mid.astype(jnp.float32)).astype(jnp.bfloat16)
    return hi, mid, lo


def _upsample_body(src_ref, m0, m1, m2, m3, wt_ref, wv_ref, o_ref):
    s4 = jax.nn.sigmoid(
        jnp.concatenate([m0[0], m1[0], m2[0], m3[0]], axis=0)
    )
    hi, mid, lo = _split3(s4)
    wt = wt_ref[...]
    sh4 = (
        jnp.dot(hi, wt, preferred_element_type=jnp.float32)
        + jnp.dot(mid, wt, preferred_element_type=jnp.float32)
    ) + jnp.dot(lo, wt, preferred_element_type=jnp.float32)

    vhi, vmid, vlo = _split3(sh4)
    one = jnp.int32(1)
    zero = jnp.int32(0)
    for j in range(MASKS_PER_STEP):
        for r in range(4):
            base = j * IN_HW + SLAB_STARTS[r]
            wv = wv_ref[r * IN_HW : (r + 1) * IN_HW, :]
            acc = (
                jnp.dot(
                    wv,
                    vhi[base : base + SLAB, :],
                    preferred_element_type=jnp.float32,
                )
                + jnp.dot(
                    wv,
                    vmid[base : base + SLAB, :],
                    preferred_element_type=jnp.float32,
                )
            ) + jnp.dot(
                wv,
                vlo[base : base + SLAB, :],
                preferred_element_type=jnp.float32,
            )
            word = jnp.where(acc[:, :IN_HW] > MASK_THR, one, zero)
            for t in range(1, 4):
                blk = acc[:, t * IN_HW : (t + 1) * IN_HW]
                word = word | jnp.where(
                    blk > MASK_THR, jnp.int32(1 << (8 * t)), zero
                )
            o_ref[j, r * IN_HW : (r + 1) * IN_HW, :] = word


@jax.jit
def kernel(cls_scores, scaled_mask_preds):
    b, q, c = cls_scores.shape
    n = q * c
    n_pad = ((n + 511) // 512) * 512
    flat = cls_scores.reshape(b, n)
    flat = jnp.pad(flat, ((0, 0), (0, n_pad - n)), constant_values=NEG_INF)

    vals, labels, src = pl.pallas_call(
        _topk_body,
        out_shape=(
            jax.ShapeDtypeStruct((b, 128), jnp.float32),
            jax.ShapeDtypeStruct((b, 128), jnp.int32),
            jax.ShapeDtypeStruct((b, 128), jnp.int32),
        ),
    )(flat)

    scores_out = vals[:, :MAX_PER_IMG]
    labels_out = labels[:, :MAX_PER_IMG]
    src_flat = src[:, :MAX_PER_IMG].reshape(b * MAX_PER_IMG)

    w = jax.image.resize(
        jnp.eye(IN_HW, dtype=jnp.float32), (ORI_H, IN_HW), method="bilinear"
    ).astype(jnp.bfloat16)
    perm = jnp.arange(ORI_W).reshape(IN_HW, 4).T.reshape(ORI_W)
    wt_perm = w.T[:, perm]
    wv = jnp.concatenate(
        [
            w[r * IN_HW : (r + 1) * IN_HW, s : s + SLAB]
            for r, s in enumerate(SLAB_STARTS)
        ],
        axis=0,
    )
    masks_flat = scaled_mask_preds.reshape(b * q, IN_HW, IN_HW)

    n_sel = b * MAX_PER_IMG
    n_steps = n_sel // MASKS_PER_STEP
    mask_spec = lambda j: pl.BlockSpec(
        (1, IN_HW, IN_HW),
        lambda i, src, j=j: (src[MASKS_PER_STEP * i + j], 0, 0),
    )
    packed = pl.pallas_call(
        _upsample_body,
        grid_spec=pltpu.PrefetchScalarGridSpec(
            num_scalar_prefetch=1,
            grid=(n_steps,),
            in_specs=[
                mask_spec(0),
                mask_spec(1),
                mask_spec(2),
                mask_spec(3),
                pl.BlockSpec((IN_HW, ORI_W), lambda i, src: (0, 0)),
                pl.BlockSpec((ORI_H, SLAB), lambda i, src: (0, 0)),
            ],
            out_specs=pl.BlockSpec(
                (MASKS_PER_STEP, ORI_H, ORI_W // 4), lambda i, src: (i, 0, 0)
            ),
        ),
        out_shape=jax.ShapeDtypeStruct((n_sel, ORI_H, ORI_W // 4), jnp.int32),
    )(src_flat, masks_flat, masks_flat, masks_flat, masks_flat, wt_perm, wv)

    bin_bytes = lax.bitcast_convert_type(packed, jnp.uint8)
    bin_masks = bin_bytes.reshape(b, MAX_PER_IMG, ORI_H, ORI_W).view(jnp.bool_)
    return scores_out, bin_masks, labels_out


# speedup vs baseline: 2.8582x; 2.8582x over previous
import functools

import jax
import jax.numpy as jnp
from jax import lax
from jax.experimental import pallas as pl
from jax.experimental.pallas import tpu as pltpu

NUM_CLASSES = 80
MAX_PER_IMG = 100
MASK_THR = 0.5
ORI_H = 512
ORI_W = 512
IN_HW = 128
MASKS_PER_STEP = 8
NEG_INF = float("-inf")


def _topk_body(scores_ref, vals_ref, labels_ref, src_ref):
    b, n = scores_ref.shape
    scores0 = scores_ref[...]
    iota = lax.broadcasted_iota(jnp.int32, (b, n), 1)
    out_iota = lax.broadcasted_iota(jnp.int32, (b, 128), 1)

    def step(k, carry):
        scores, vals_acc, idx_acc = carry
        m = jnp.max(scores, axis=1, keepdims=True)
        cand = jnp.where(scores == m, iota, jnp.int32(2**30))
        idx = jnp.min(cand, axis=1, keepdims=True)
        vals_acc = jnp.where(out_iota == k, m, vals_acc)
        idx_acc = jnp.where(out_iota == k, idx, idx_acc)
        scores = jnp.where(iota == idx, NEG_INF, scores)
        return scores, vals_acc, idx_acc

    init = (
        scores0,
        jnp.zeros((b, 128), jnp.float32),
        jnp.zeros((b, 128), jnp.int32),
    )
    _, vals_acc, idx_acc = lax.fori_loop(0, MAX_PER_IMG, step, init)

    row = lax.broadcasted_iota(jnp.int32, (b, 128), 0)
    vals_ref[...] = vals_acc
    labels_ref[...] = idx_acc % NUM_CLASSES
    src_ref[...] = idx_acc // NUM_CLASSES + MAX_PER_IMG * row


def _split3(a):
    hi = a.astype(jnp.bfloat16)
    r1 = a - hi.astype(jnp.float32)
    mid = r1.astype(jnp.bfloat16)
    lo = (r1 - mid.astype(jnp.float32)).astype(jnp.bfloat16)
    return hi, mid, lo


SLAB = 64
SLAB_STARTS = (0, 16, 48, 64)


def _upsample_body(src_ref, *refs):
    mrefs = refs[:MASKS_PER_STEP]
    wt_ref, wv_ref, o_ref = refs[MASKS_PER_STEP:]
    s4 = jax.nn.sigmoid(jnp.concatenate([m[0] for m in mrefs], axis=0))
    hi, mid, lo = _split3(s4)
    wt = wt_ref[...]
    sh4 = (
        jnp.dot(hi, wt, preferred_element_type=jnp.float32)
        + jnp.dot(mid, wt, preferred_element_type=jnp.float32)
    ) + jnp.dot(lo, wt, preferred_element_type=jnp.float32)

    vhi, vmid, vlo = _split3(sh4)
    for j in range(MASKS_PER_STEP):
        for r in range(4):
            base = j * IN_HW + SLAB_STARTS[r]
            wv = wv_ref[r * IN_HW : (r + 1) * IN_HW, :]
            acc = (
                jnp.dot(
                    wv,
                    vhi[base : base + SLAB, :],
                    preferred_element_type=jnp.float32,
                )
                + jnp.dot(
                    wv,
                    vmid[base : base + SLAB, :],
                    preferred_element_type=jnp.float32,
                )
            ) + jnp.dot(
                wv,
                vlo[base : base + SLAB, :],
                preferred_element_type=jnp.float32,
            )
            o_ref[j, r * IN_HW : (r + 1) * IN_HW, :] = acc > MASK_THR


@jax.jit
def kernel(cls_scores, scaled_mask_preds):
    b, q, c = cls_scores.shape
    n = q * c
    n_pad = ((n + 511) // 512) * 512
    flat = cls_scores.reshape(b, n)
    flat = jnp.pad(flat, ((0, 0), (0, n_pad - n)), constant_values=NEG_INF)

    vals, labels, src = pl.pallas_call(
        _topk_body,
        out_shape=(
            jax.ShapeDtypeStruct((b, 128), jnp.float32),
            jax.ShapeDtypeStruct((b, 128), jnp.int32),
            jax.ShapeDtypeStruct((b, 128), jnp.int32),
        ),
    )(flat)

    scores_out = vals[:, :MAX_PER_IMG]
    labels_out = labels[:, :MAX_PER_IMG]
    src_flat = src[:, :MAX_PER_IMG].reshape(b * MAX_PER_IMG)

    w = jax.image.resize(
        jnp.eye(IN_HW, dtype=jnp.float32), (ORI_H, IN_HW), method="bilinear"
    ).astype(jnp.bfloat16)
    wv = jnp.concatenate(
        [
            w[r * IN_HW : (r + 1) * IN_HW, s : s + SLAB]
            for r, s in enumerate(SLAB_STARTS)
        ],
        axis=0,
    )
    masks_flat = scaled_mask_preds.reshape(b * q, IN_HW, IN_HW)

    n_sel = b * MAX_PER_IMG
    n_steps = n_sel // MASKS_PER_STEP
    mask_spec = lambda j: pl.BlockSpec(
        (1, IN_HW, IN_HW),
        lambda i, src, j=j: (src[MASKS_PER_STEP * i + j], 0, 0),
    )
    bin_masks = pl.pallas_call(
        _upsample_body,
        grid_spec=pltpu.PrefetchScalarGridSpec(
            num_scalar_prefetch=1,
            grid=(n_steps,),
            in_specs=[
                *[mask_spec(j) for j in range(MASKS_PER_STEP)],
                pl.BlockSpec((IN_HW, ORI_W), lambda i, src: (0, 0)),
                pl.BlockSpec((ORI_H, SLAB), lambda i, src: (0, 0)),
            ],
            out_specs=pl.BlockSpec(
                (MASKS_PER_STEP, ORI_H, ORI_W), lambda i, src: (i, 0, 0)
            ),
        ),
        out_shape=jax.ShapeDtypeStruct((n_sel, ORI_H, ORI_W), jnp.bool_),
    )(src_flat, *([masks_flat] * MASKS_PER_STEP), w.T, wv)

    return scores_out, bin_masks.reshape(b, MAX_PER_IMG, ORI_H, ORI_W), labels_out


# speedup vs baseline: 3.3013x; 1.1550x over previous
import functools

import jax
import jax.numpy as jnp
from jax import lax
from jax.experimental import pallas as pl
from jax.experimental.pallas import tpu as pltpu

NUM_CLASSES = 80
MAX_PER_IMG = 100
MASK_THR = 0.5
ORI_H = 512
ORI_W = 512
IN_HW = 128
MASKS_PER_STEP = 8
NEG_INF = float("-inf")


def _topk_body(scores_ref, vals_ref, labels_ref, src_ref):
    b, n = scores_ref.shape
    scores0 = scores_ref[...]
    iota = lax.broadcasted_iota(jnp.int32, (b, n), 1)
    out_iota = lax.broadcasted_iota(jnp.int32, (b, 128), 1)

    def step(k, carry):
        scores, vals_acc, idx_acc = carry
        m = jnp.max(scores, axis=1, keepdims=True)
        cand = jnp.where(scores == m, iota, jnp.int32(2**30))
        idx = jnp.min(cand, axis=1, keepdims=True)
        vals_acc = jnp.where(out_iota == k, m, vals_acc)
        idx_acc = jnp.where(out_iota == k, idx, idx_acc)
        scores = jnp.where(iota == idx, NEG_INF, scores)
        return scores, vals_acc, idx_acc

    init = (
        scores0,
        jnp.zeros((b, 128), jnp.float32),
        jnp.zeros((b, 128), jnp.int32),
    )
    _, vals_acc, idx_acc = lax.fori_loop(0, MAX_PER_IMG, step, init)

    row = lax.broadcasted_iota(jnp.int32, (b, 128), 0)
    vals_ref[...] = vals_acc
    labels_ref[...] = idx_acc % NUM_CLASSES
    src_ref[...] = idx_acc // NUM_CLASSES + MAX_PER_IMG * row


def _split2(a):
    hi = a.astype(jnp.bfloat16)
    mid = (a - hi.astype(jnp.float32)).astype(jnp.bfloat16)
    return hi, mid


def _split3(a):
    hi = a.astype(jnp.bfloat16)
    r1 = a - hi.astype(jnp.float32)
    mid = r1.astype(jnp.bfloat16)
    lo = (r1 - mid.astype(jnp.float32)).astype(jnp.bfloat16)
    return hi, mid, lo


SLAB = 64
SLAB_STARTS = (0, 16, 48, 64)


def _upsample_body(src_ref, *refs):
    mrefs = refs[:MASKS_PER_STEP]
    wt_ref, wv_ref, o_ref = refs[MASKS_PER_STEP:]
    s4 = jax.nn.sigmoid(jnp.concatenate([m[0] for m in mrefs], axis=0))
    hi, mid, lo = _split3(s4)
    wt = wt_ref[...]
    sh4 = (
        jnp.dot(hi, wt, preferred_element_type=jnp.float32)
        + jnp.dot(mid, wt, preferred_element_type=jnp.float32)
    ) + jnp.dot(lo, wt, preferred_element_type=jnp.float32)

    vhi, vmid = _split2(sh4)
    for j in range(MASKS_PER_STEP):
        for r in range(4):
            base = j * IN_HW + SLAB_STARTS[r]
            wv = wv_ref[r * IN_HW : (r + 1) * IN_HW, :]
            acc = jnp.dot(
                wv,
                vhi[base : base + SLAB, :],
                preferred_element_type=jnp.float32,
            ) + jnp.dot(
                wv,
                vmid[base : base + SLAB, :],
                preferred_element_type=jnp.float32,
            )
            o_ref[j, r * IN_HW : (r + 1) * IN_HW, :] = acc > MASK_THR


@jax.jit
def kernel(cls_scores, scaled_mask_preds):
    b, q, c = cls_scores.shape
    n = q * c
    n_pad = ((n + 511) // 512) * 512
    flat = cls_scores.reshape(b, n)
    flat = jnp.pad(flat, ((0, 0), (0, n_pad - n)), constant_values=NEG_INF)

    vals, labels, src = pl.pallas_call(
        _topk_body,
        out_shape=(
            jax.ShapeDtypeStruct((b, 128), jnp.float32),
            jax.ShapeDtypeStruct((b, 128), jnp.int32),
            jax.ShapeDtypeStruct((b, 128), jnp.int32),
        ),
    )(flat)

    scores_out = vals[:, :MAX_PER_IMG]
    labels_out = labels[:, :MAX_PER_IMG]
    src_flat = src[:, :MAX_PER_IMG].reshape(b * MAX_PER_IMG)

    w = jax.image.resize(
        jnp.eye(IN_HW, dtype=jnp.float32), (ORI_H, IN_HW), method="bilinear"
    ).astype(jnp.bfloat16)
    wv = jnp.concatenate(
        [
            w[r * IN_HW : (r + 1) * IN_HW, s : s + SLAB]
            for r, s in enumerate(SLAB_STARTS)
        ],
        axis=0,
    )
    masks_flat = scaled_mask_preds.reshape(b * q, IN_HW, IN_HW)

    n_sel = b * MAX_PER_IMG
    n_steps = n_sel // MASKS_PER_STEP
    mask_spec = lambda j: pl.BlockSpec(
        (1, IN_HW, IN_HW),
        lambda i, src, j=j: (src[MASKS_PER_STEP * i + j], 0, 0),
    )
    bin_masks = pl.pallas_call(
        _upsample_body,
        grid_spec=pltpu.PrefetchScalarGridSpec(
            num_scalar_prefetch=1,
            grid=(n_steps,),
            in_specs=[
                *[mask_spec(j) for j in range(MASKS_PER_STEP)],
                pl.BlockSpec((IN_HW, ORI_W), lambda i, src: (0, 0)),
                pl.BlockSpec((ORI_H, SLAB), lambda i, src: (0, 0)),
            ],
            out_specs=pl.BlockSpec(
                (MASKS_PER_STEP, ORI_H, ORI_W), lambda i, src: (i, 0, 0)
            ),
        ),
        out_shape=jax.ShapeDtypeStruct((n_sel, ORI_H, ORI_W), jnp.bool_),
    )(src_flat, *([masks_flat] * MASKS_PER_STEP), w.T, wv)

    return scores_out, bin_masks.reshape(b, MAX_PER_IMG, ORI_H, ORI_W), labels_out


# speedup vs baseline: 3.4016x; 1.0304x over previous
import functools

import jax
import jax.numpy as jnp
from jax import lax
from jax.experimental import pallas as pl
from jax.experimental.pallas import tpu as pltpu

NUM_CLASSES = 80
MAX_PER_IMG = 100
MASK_THR = 0.5
ORI_H = 512
ORI_W = 512
IN_HW = 128
MASKS_PER_STEP = 8
NEG_INF = float("-inf")


def _topk_body(scores_ref, vals_ref, labels_ref, src_ref):
    b, n = scores_ref.shape
    scores0 = scores_ref[...]
    iota = lax.broadcasted_iota(jnp.int32, (b, n), 1)
    out_iota = lax.broadcasted_iota(jnp.int32, (b, 128), 1)

    def step(k, carry):
        scores, vals_acc, idx_acc = carry
        m = jnp.max(scores, axis=1, keepdims=True)
        cand = jnp.where(scores == m, iota, jnp.int32(2**30))
        idx = jnp.min(cand, axis=1, keepdims=True)
        vals_acc = jnp.where(out_iota == k, m, vals_acc)
        idx_acc = jnp.where(out_iota == k, idx, idx_acc)
        scores = jnp.where(iota == idx, NEG_INF, scores)
        return scores, vals_acc, idx_acc

    init = (
        scores0,
        jnp.zeros((b, 128), jnp.float32),
        jnp.zeros((b, 128), jnp.int32),
    )
    _, vals_acc, idx_acc = lax.fori_loop(0, MAX_PER_IMG, step, init)

    row = lax.broadcasted_iota(jnp.int32, (b, 128), 0)
    vals_ref[...] = vals_acc
    labels_ref[...] = idx_acc % NUM_CLASSES
    src_ref[...] = idx_acc // NUM_CLASSES + MAX_PER_IMG * row


def _split2(a):
    hi = a.astype(jnp.bfloat16)
    mid = (a - hi.astype(jnp.float32)).astype(jnp.bfloat16)
    return hi, mid


def _split3(a):
    hi = a.astype(jnp.bfloat16)
    r1 = a - hi.astype(jnp.float32)
    mid = r1.astype(jnp.bfloat16)
    lo = (r1 - mid.astype(jnp.float32)).astype(jnp.bfloat16)
    return hi, mid, lo


SLAB = 64
SLAB_STARTS = (0, 16, 48, 64)


def _upsample_body(src_ref, *refs):
    mrefs = refs[:MASKS_PER_STEP]
    wt_ref, wv_ref, o_ref = refs[MASKS_PER_STEP:]
    s4 = jax.nn.sigmoid(jnp.concatenate([m[0] for m in mrefs], axis=0))
    hi, mid = _split2(s4)
    wt = wt_ref[...]
    sh4 = jnp.dot(hi, wt, preferred_element_type=jnp.float32) + jnp.dot(
        mid, wt, preferred_element_type=jnp.float32
    )

    vhi, vmid = _split2(sh4)
    for j in range(MASKS_PER_STEP):
        for r in range(4):
            base = j * IN_HW + SLAB_STARTS[r]
            wv = wv_ref[r * IN_HW : (r + 1) * IN_HW, :]
            acc = jnp.dot(
                wv,
                vhi[base : base + SLAB, :],
                preferred_element_type=jnp.float32,
            ) + jnp.dot(
                wv,
                vmid[base : base + SLAB, :],
                preferred_element_type=jnp.float32,
            )
            o_ref[j, r * IN_HW : (r + 1) * IN_HW, :] = acc > MASK_THR


@jax.jit
def kernel(cls_scores, scaled_mask_preds):
    b, q, c = cls_scores.shape
    n = q * c
    n_pad = ((n + 511) // 512) * 512
    flat = cls_scores.reshape(b, n)
    flat = jnp.pad(flat, ((0, 0), (0, n_pad - n)), constant_values=NEG_INF)

    vals, labels, src = pl.pallas_call(
        _topk_body,
        out_shape=(
            jax.ShapeDtypeStruct((b, 128), jnp.float32),
            jax.ShapeDtypeStruct((b, 128), jnp.int32),
            jax.ShapeDtypeStruct((b, 128), jnp.int32),
        ),
    )(flat)

    scores_out = vals[:, :MAX_PER_IMG]
    labels_out = labels[:, :MAX_PER_IMG]
    src_flat = src[:, :MAX_PER_IMG].reshape(b * MAX_PER_IMG)

    w = jax.image.resize(
        jnp.eye(IN_HW, dtype=jnp.float32), (ORI_H, IN_HW), method="bilinear"
    ).astype(jnp.bfloat16)
    wv = jnp.concatenate(
        [
            w[r * IN_HW : (r + 1) * IN_HW, s : s + SLAB]
            for r, s in enumerate(SLAB_STARTS)
        ],
        axis=0,
    )
    masks_flat = scaled_mask_preds.reshape(b * q, IN_HW, IN_HW)

    n_sel = b * MAX_PER_IMG
    n_steps = n_sel // MASKS_PER_STEP
    mask_spec = lambda j: pl.BlockSpec(
        (1, IN_HW, IN_HW),
        lambda i, src, j=j: (src[MASKS_PER_STEP * i + j], 0, 0),
    )
    bin_masks = pl.pallas_call(
        _upsample_body,
        grid_spec=pltpu.PrefetchScalarGridSpec(
            num_scalar_prefetch=1,
            grid=(n_steps,),
            in_specs=[
                *[mask_spec(j) for j in range(MASKS_PER_STEP)],
                pl.BlockSpec((IN_HW, ORI_W), lambda i, src: (0, 0)),
                pl.BlockSpec((ORI_H, SLAB), lambda i, src: (0, 0)),
            ],
            out_specs=pl.BlockSpec(
                (MASKS_PER_STEP, ORI_H, ORI_W), lambda i, src: (i, 0, 0)
            ),
        ),
        out_shape=jax.ShapeDtypeStruct((n_sel, ORI_H, ORI_W), jnp.bool_),
    )(src_flat, *([masks_flat] * MASKS_PER_STEP), w.T, wv)

    return scores_out, bin_masks.reshape(b, MAX_PER_IMG, ORI_H, ORI_W), labels_out


# speedup vs baseline: 3.4024x; 1.0002x over previous
import jax
import jax.numpy as jnp
from jax import lax
from jax.experimental import pallas as pl
from jax.experimental.pallas import tpu as pltpu

NUM_CLASSES = 80
MAX_PER_IMG = 100
MASK_THR = 0.5
ORI_H = 512
ORI_W = 512
IN_HW = 128
MASKS_PER_STEP = 8
NEG_INF = float("-inf")


def _topk_body(scores_ref, vals_ref, labels_ref, src_ref):
    b, n = scores_ref.shape
    scores0 = scores_ref[...]
    iota = lax.broadcasted_iota(jnp.int32, (b, n), 1)
    out_iota = lax.broadcasted_iota(jnp.int32, (b, 128), 1)

    def step(k, carry):
        scores, vals_acc, idx_acc = carry
        m = jnp.max(scores, axis=1, keepdims=True)
        cand = jnp.where(scores == m, iota, jnp.int32(2**30))
        idx = jnp.min(cand, axis=1, keepdims=True)
        vals_acc = jnp.where(out_iota == k, m, vals_acc)
        idx_acc = jnp.where(out_iota == k, idx, idx_acc)
        scores = jnp.where(iota == idx, NEG_INF, scores)
        return scores, vals_acc, idx_acc

    init = (
        scores0,
        jnp.zeros((b, 128), jnp.float32),
        jnp.zeros((b, 128), jnp.int32),
    )
    _, vals_acc, idx_acc = lax.fori_loop(0, MAX_PER_IMG, step, init)

    row = lax.broadcasted_iota(jnp.int32, (b, 128), 0)
    vals_ref[...] = vals_acc
    labels_ref[...] = idx_acc % NUM_CLASSES
    src_ref[...] = idx_acc // NUM_CLASSES + MAX_PER_IMG * row


def _split2(a):
    hi = a.astype(jnp.bfloat16)
    mid = (a - hi.astype(jnp.float32)).astype(jnp.bfloat16)
    return hi, mid


SLAB = 64
SLAB_STARTS = (0, 16, 48, 64)


def _upsample_body(src_ref, *refs):
    mrefs = refs[:MASKS_PER_STEP]
    wt_ref, wv_ref, o_ref = refs[MASKS_PER_STEP:]
    s4 = jax.nn.sigmoid(jnp.concatenate([m[0] for m in mrefs], axis=0))
    hi, mid = _split2(s4)
    wt = wt_ref[...]
    sh4 = jnp.dot(hi, wt, preferred_element_type=jnp.float32) + jnp.dot(
        mid, wt, preferred_element_type=jnp.float32
    )

    vhi, vmid = _split2(sh4)
    for j in range(MASKS_PER_STEP):
        for r in range(4):
            base = j * IN_HW + SLAB_STARTS[r]
            wv = wv_ref[r * IN_HW : (r + 1) * IN_HW, :]
            acc = jnp.dot(
                wv,
                vhi[base : base + SLAB, :],
                preferred_element_type=jnp.float32,
            ) + jnp.dot(
                wv,
                vmid[base : base + SLAB, :],
                preferred_element_type=jnp.float32,
            )
            o_ref[j, r * IN_HW : (r + 1) * IN_HW, :] = acc > MASK_THR


@jax.jit
def kernel(cls_scores, scaled_mask_preds):
    b, q, c = cls_scores.shape
    n = q * c
    n_pad = ((n + 511) // 512) * 512
    flat = cls_scores.reshape(b, n)
    flat = jnp.pad(flat, ((0, 0), (0, n_pad - n)), constant_values=NEG_INF)

    vals, labels, src = pl.pallas_call(
        _topk_body,
        out_shape=(
            jax.ShapeDtypeStruct((b, 128), jnp.float32),
            jax.ShapeDtypeStruct((b, 128), jnp.int32),
            jax.ShapeDtypeStruct((b, 128), jnp.int32),
        ),
    )(flat)

    scores_out = vals[:, :MAX_PER_IMG]
    labels_out = labels[:, :MAX_PER_IMG]
    src_flat = src[:, :MAX_PER_IMG].reshape(b * MAX_PER_IMG)

    w = jax.image.resize(
        jnp.eye(IN_HW, dtype=jnp.float32), (ORI_H, IN_HW), method="bilinear"
    ).astype(jnp.bfloat16)
    wv = jnp.concatenate(
        [
            w[r * IN_HW : (r + 1) * IN_HW, s : s + SLAB]
            for r, s in enumerate(SLAB_STARTS)
        ],
        axis=0,
    )
    masks_flat = scaled_mask_preds.reshape(b * q, IN_HW, IN_HW)

    n_sel = b * MAX_PER_IMG
    n_steps = n_sel // MASKS_PER_STEP
    mask_spec = lambda j: pl.BlockSpec(
        (1, IN_HW, IN_HW),
        lambda i, src, j=j: (src[MASKS_PER_STEP * i + j], 0, 0),
    )
    bin_masks = pl.pallas_call(
        _upsample_body,
        grid_spec=pltpu.PrefetchScalarGridSpec(
            num_scalar_prefetch=1,
            grid=(n_steps,),
            in_specs=[
                *[mask_spec(j) for j in range(MASKS_PER_STEP)],
                pl.BlockSpec((IN_HW, ORI_W), lambda i, src: (0, 0)),
                pl.BlockSpec((ORI_H, SLAB), lambda i, src: (0, 0)),
            ],
            out_specs=pl.BlockSpec(
                (MASKS_PER_STEP, ORI_H, ORI_W), lambda i, src: (i, 0, 0)
            ),
        ),
        out_shape=jax.ShapeDtypeStruct((n_sel, ORI_H, ORI_W), jnp.bool_),
    )(src_flat, *([masks_flat] * MASKS_PER_STEP), w.T, wv)

    return scores_out, bin_masks.reshape(b, MAX_PER_IMG, ORI_H, ORI_W), labels_out
